# baseline (speedup 1.0000x reference)
import functools

import jax
import jax.numpy as jnp
from jax import lax
from jax.experimental import pallas as pl
from jax.experimental.pallas import tpu as pltpu
from jax.experimental.pallas import tpu_sc as plsc

_D = 64
_BB = 128
_T = 200


@functools.cache
def _build(n_b: int, n_t: int, d: int):
    info = plsc.get_sparse_core_info()
    nc = info.num_cores
    nw = nc * info.num_subcores
    assert n_b == nw * _BB and n_t == _T and d == _D

    mesh = plsc.VectorSubcoreMesh(core_axis_name="c", subcore_axis_name="s")

    @functools.partial(
        pl.kernel,
        mesh=mesh,
        out_type=jax.ShapeDtypeStruct((_T, d // 8, n_b // 128, 8, 128),
                                      jnp.float32),
        compiler_params=pltpu.CompilerParams(
            use_tc_tiling_on_sc=False, needs_layout_passes=False
        ),
        scratch_types=[
            pltpu.VMEM((_T, _BB), jnp.int32),
            pltpu.VMEM((_BB, d), jnp.float32),
            pltpu.VMEM((_BB, d), jnp.float32),
            pltpu.VMEM((d // 8, 8, 128), jnp.float32),
            pltpu.SemaphoreType.DMA,
            pltpu.SemaphoreType.DMA,
        ],
    )
    def k(xt_hbm, table_hbm, out_hbm, idx_v, rows0, rows1, tr_v, g0, g1):
        wid = lax.axis_index("s") * nc + lax.axis_index("c")
        pltpu.sync_copy(xt_hbm.at[:, pl.ds(wid * _BB, _BB)], idx_v)

        lanes = jnp.arange(16, dtype=jnp.int32)

        def fire_g(t, rows, gsem):
            pltpu.async_copy(table_hbm.at[idx_v.at[t]], rows, gsem)

        def wait_g(rows, gsem):
            pltpu.make_async_copy(table_hbm.at[pl.ds(0, _BB)], rows, gsem).wait()

        def step(t, rows):
            for dd in range(d // 8):
                for s in range(8):
                    dcol = jnp.full((16,), dd * 8 + s, dtype=jnp.int32)
                    for g in range(8):
                        v = plsc.load_gather(rows, [lanes + 16 * g, dcol])
                        tr_v[dd, s, pl.ds(16 * g, 16)] = v
            pltpu.sync_copy(tr_v, out_hbm.at[t, :, wid])

        fire_g(0, rows0, g0)
        fire_g(1, rows1, g1)

        def body(i, carry):
            t0 = 2 * i
            wait_g(rows0, g0)
            step(t0, rows0)
            fire_g(t0 + 2, rows0, g0)
            wait_g(rows1, g1)
            step(t0 + 1, rows1)
            fire_g(t0 + 3, rows1, g1)
            return carry

        lax.fori_loop(0, _T // 2 - 1, body, 0)

        wait_g(rows0, g0)
        step(_T - 2, rows0)
        wait_g(rows1, g1)
        step(_T - 1, rows1)

    return k


def kernel(x, table):
    s0, s1 = x.shape
    out5 = _build(s0, s1, _D)(x.T, table)
    return out5.transpose(2, 4, 0, 1, 3).reshape(s0, s1, _D)


# speedup vs baseline: 1.6692x; 1.6692x over previous
import functools

import jax
import jax.numpy as jnp
from jax import lax
from jax.experimental import pallas as pl
from jax.experimental.pallas import tpu as pltpu
from jax.experimental.pallas import tpu_sc as plsc

_D = 64
_CHUNK = 512
_GPS = 1
_STEP = _CHUNK * _GPS


@functools.cache
def _build(n_idx_rows: int, chunk: int, d: int):
    info = plsc.get_sparse_core_info()
    nw = info.num_cores * info.num_subcores
    rows_per_w = n_idx_rows // nw
    n_steps = rows_per_w // _GPS
    b = n_idx_rows * chunk

    mesh = plsc.VectorSubcoreMesh(core_axis_name="c", subcore_axis_name="s")

    @functools.partial(
        pl.kernel,
        mesh=mesh,
        out_type=jax.ShapeDtypeStruct((b, d), jnp.float32),
        compiler_params=pltpu.CompilerParams(use_tc_tiling_on_sc=False),
        scratch_types=[
            pltpu.VMEM((rows_per_w, chunk), jnp.int32),
            pltpu.VMEM((_STEP, d), jnp.float32),
            pltpu.VMEM((_STEP, d), jnp.float32),
            pltpu.SemaphoreType.DMA,
            pltpu.SemaphoreType.DMA,
            pltpu.SemaphoreType.DMA,
            pltpu.SemaphoreType.DMA,
        ],
    )
    def k(x_hbm, table_hbm, out_hbm, idx_v, rows0, rows1, g0, g1, o0, o1):
        wid = lax.axis_index("s") * info.num_cores + lax.axis_index("c")
        pltpu.sync_copy(x_hbm.at[pl.ds(wid * rows_per_w, rows_per_w)], idx_v)
        base = wid * rows_per_w * chunk
        bufs = ((rows0, g0, o0), (rows1, g1, o1))

        def fire_g(s, bi):
            rows, gsem, _ = bufs[bi]
            for g in range(_GPS):
                pltpu.async_copy(
                    table_hbm.at[idx_v.at[s * _GPS + g]],
                    rows.at[pl.ds(g * chunk, chunk)],
                    gsem,
                )

        def wait_g(bi):
            rows, gsem, _ = bufs[bi]
            pltpu.make_async_copy(out_hbm.at[pl.ds(0, _STEP)], rows, gsem).wait()

        def fire_o(s, bi):
            rows, _, osem = bufs[bi]
            pltpu.async_copy(rows, out_hbm.at[pl.ds(base + s * _STEP, _STEP)], osem)

        def wait_o(bi):
            rows, _, osem = bufs[bi]
            pltpu.make_async_copy(rows, out_hbm.at[pl.ds(0, _STEP)], osem).wait()

        fire_g(0, 0)
        fire_g(1, 1)

        def body(i, c):
            s0 = 2 * i
            wait_g(0)
            fire_o(s0, 0)
            wait_g(1)
            fire_o(s0 + 1, 1)
            wait_o(0)
            fire_g(s0 + 2, 0)
            wait_o(1)
            fire_g(s0 + 3, 1)
            return c

        lax.fori_loop(0, n_steps // 2 - 1, body, 0)

        wait_g(0)
        fire_o(n_steps - 2, 0)
        wait_g(1)
        fire_o(n_steps - 1, 1)
        wait_o(0)
        wait_o(1)

    return k


def kernel(x, table):
    s0, s1 = x.shape
    x2d = x.reshape(-1, _CHUNK)
    out = _build(x2d.shape[0], _CHUNK, _D)(x2d, table)
    return out.reshape(s0, s1, _D)
